# baseline (speedup 1.0000x reference)
import functools

import jax
import jax.numpy as jnp
from jax import lax
from jax.experimental import pallas as pl
from jax.experimental.pallas import tpu as pltpu
from jax.experimental.pallas import tpu_sc as plsc

_LANES = 16
_SEG = 25
_NBUF = 8
_PCH = 625


@functools.lru_cache(maxsize=None)
def _build_pack(V, D):
  DW = D // 2
  NC, NS = 2, 16
  NW = NC * NS
  RPW = V // NW
  NCH = RPW // _PCH
  assert RPW % _PCH == 0 and NCH % 2 == 0
  K2 = DW // _LANES

  mesh = plsc.VectorSubcoreMesh(core_axis_name="c", subcore_axis_name="s")

  @functools.partial(
      pl.kernel,
      out_type=jax.ShapeDtypeStruct((V, DW), jnp.int32),
      mesh=mesh,
      compiler_params=pltpu.CompilerParams(use_tc_tiling_on_sc=False),
      scratch_types=[
          pltpu.VMEM((2 * _PCH, D), jnp.float32),
          pltpu.VMEM((2 * _PCH, DW), jnp.int32),
      ] + [pltpu.SemaphoreType.DMA] * 4,
  )
  def pack_kernel(tab_hbm, out_hbm, in_v, out_v, *sems):
    wid = lax.axis_index("s") * NC + lax.axis_index("c")
    base = wid * RPW
    isems, osems = sems[:2], sems[2:]
    half = jnp.full((_LANES,), 0x8000, jnp.int32)
    maskh = jnp.full((_LANES,), -65536, jnp.int32)

    def fire_in(c, b):
      pltpu.async_copy(tab_hbm.at[pl.ds(base + c * _PCH, _PCH)],
                       in_v.at[pl.ds(b * _PCH, _PCH)], isems[b])

    def wait_in(b):
      pltpu.make_async_copy(tab_hbm.at[pl.ds(0, _PCH)],
                            in_v.at[pl.ds(b * _PCH, _PCH)], isems[b]).wait()

    def wait_out(b):
      pltpu.make_async_copy(out_hbm.at[pl.ds(0, _PCH)],
                            out_v.at[pl.ds(b * _PCH, _PCH)], osems[b]).wait()

    def pack(c, b):
      def row_body(i, carry):
        src = b * _PCH + i
        for k2 in range(K2):
          lo = lax.bitcast_convert_type(
              in_v[src, pl.ds(k2 * _LANES, _LANES)], jnp.int32)
          hi = lax.bitcast_convert_type(
              in_v[src, pl.ds(DW + k2 * _LANES, _LANES)], jnp.int32)
          w = (lax.shift_right_logical(lo + half, 16)) | ((hi + half) & maskh)
          out_v[src, pl.ds(k2 * _LANES, _LANES)] = w
        return carry

      lax.fori_loop(0, _PCH, row_body, 0)
      pltpu.async_copy(out_v.at[pl.ds(b * _PCH, _PCH)],
                       out_hbm.at[pl.ds(base + c * _PCH, _PCH)], osems[b])

    fire_in(0, 0)

    def body(h, carry):
      for b in range(2):
        c = 2 * h + b

        @pl.when(c + 1 < NCH)
        def _():
          fire_in(c + 1, 1 - b)

        wait_in(b)

        @pl.when(c >= 2)
        def _():
          wait_out(b)

        pack(c, b)
      return carry

    lax.fori_loop(0, NCH // 2, body, 0)
    wait_out(0)
    wait_out(1)

  return pack_kernel


@functools.lru_cache(maxsize=None)
def _build(B, L, V, D, C):
  assert D % (2 * _LANES) == 0
  KD = D // _LANES
  DW = D // 2
  NC, NS = 2, 16
  NW = NC * NS
  assert B % NW == 0
  BPW = B // NW
  assert BPW % _NBUF == 0
  assert L % _SEG == 0
  NSEG = L // _SEG
  inv_l = 1.0 / L
  UNROLL = 25
  assert L % UNROLL == 0

  mesh = plsc.VectorSubcoreMesh(core_axis_name="c", subcore_axis_name="s")

  @functools.partial(
      pl.kernel,
      out_type=jax.ShapeDtypeStruct((B, _LANES), jnp.float32),
      mesh=mesh,
      compiler_params=pltpu.CompilerParams(use_tc_tiling_on_sc=False),
      scratch_types=[
          pltpu.VMEM((BPW * NSEG, _SEG), jnp.int32),
          pltpu.VMEM((_NBUF * L, DW), jnp.int32),
          pltpu.VMEM((D, _LANES), jnp.float32),
          pltpu.VMEM((_LANES,), jnp.float32),
          pltpu.VMEM((BPW, _LANES), jnp.float32),
      ] + [pltpu.SemaphoreType.DMA] * _NBUF,
  )
  def sc_kernel(text_hbm, table_hbm, fcwt_hbm, fcb_hbm, out_hbm,
                idx_v, rows_v, fcwt_v, fcb_v, out_v, *sems):
    wid = lax.axis_index("s") * NC + lax.axis_index("c")
    base = wid * BPW

    pltpu.sync_copy(fcwt_hbm, fcwt_v)
    pltpu.sync_copy(fcb_hbm, fcb_v)
    pltpu.sync_copy(text_hbm.at[pl.ds(base * NSEG, BPW * NSEG)], idx_v)
    fcb_vec = fcb_v[pl.ds(0, _LANES)]
    zero = jnp.zeros((_LANES,), jnp.float32)

    def fire(r, b):
      for j in range(NSEG):
        pltpu.async_copy(
            table_hbm.at[idx_v.at[r * NSEG + j]],
            rows_v.at[pl.ds(b * L + j * _SEG, _SEG)], sems[b])

    def drain(b):
      pltpu.make_async_copy(table_hbm.at[pl.ds(0, L)],
                            rows_v.at[pl.ds(b * L, L)], sems[b]).wait()

    def process(r, b):
      off = b * L
      mask_hi = jnp.full((_LANES,), -65536, jnp.int32)

      def red_body(t, accs):
        ib = off + t * UNROLL
        accs = list(accs)
        for u in range(UNROLL):
          for k2 in range(DW // _LANES):
            w = rows_v[ib + u, pl.ds(k2 * _LANES, _LANES)]
            lo = lax.bitcast_convert_type(w << 16, jnp.float32)
            hi = lax.bitcast_convert_type(w & mask_hi, jnp.float32)
            accs[2 * k2] = accs[2 * k2] + lo
            accs[2 * k2 + 1] = accs[2 * k2 + 1] + hi
        return tuple(accs)

      accs = lax.fori_loop(0, L // UNROLL, red_body, (zero,) * KD)
      pooled = [a * inv_l for a in accs]

      parts = [fcb_vec, zero, zero, zero]
      for d in range(D):
        half, dd = divmod(d, D // 2)
        vreg = 2 * (dd // _LANES) + half
        lane = dd % _LANES
        parts[d % 4] = parts[d % 4] + (
            pooled[vreg][lane] * fcwt_v[d, pl.ds(0, _LANES)])
      out_row = (parts[0] + parts[1]) + (parts[2] + parts[3])
      out_v[r, pl.ds(0, _LANES)] = out_row

    for b in range(_NBUF - 1):
      fire(b, b)

    def body(h, carry):
      for b in range(_NBUF):
        r = h * _NBUF + b
        nxt = r + _NBUF - 1
        pb = (b - 1) % _NBUF

        @pl.when(nxt < BPW)
        def _():
          fire(nxt, pb)

        drain(b)
        process(r, b)
      return carry

    lax.fori_loop(0, BPW // _NBUF, body, 0)
    pltpu.sync_copy(out_v, out_hbm.at[pl.ds(base, BPW)])

  return sc_kernel


def kernel(text, emb_table, fc_w, fc_b):
  B, L = text.shape
  V, D = emb_table.shape
  C = fc_w.shape[0]
  text = text.astype(jnp.int32).reshape(B * (L // _SEG), _SEG)
  table_packed = _build_pack(V, D)(emb_table)
  fcwt = jnp.zeros((D, _LANES), jnp.float32).at[:, :C].set(fc_w.T)
  fcb_pad = jnp.zeros((_LANES,), jnp.float32).at[:C].set(fc_b)
  out = _build(B, L, V, D, C)(text, table_packed, fcwt, fcb_pad)
  return out[:, :C]


# speedup vs baseline: 1.5616x; 1.5616x over previous
import functools

import jax
import jax.numpy as jnp
from jax import lax
from jax.experimental import pallas as pl
from jax.experimental.pallas import tpu as pltpu
from jax.experimental.pallas import tpu_sc as plsc

_LANES = 16
_IDXW = 100
_NBUF = 4


@functools.lru_cache(maxsize=None)
def _build(B, L, V, D, C):
  assert D % _LANES == 0
  KD = D // _LANES
  NC, NS = 2, 16
  NW = NC * NS
  assert B % NW == 0
  BPW = B // NW
  assert BPW % _NBUF == 0
  SPLITS = L // _IDXW
  assert L % _IDXW == 0
  inv_l = 1.0 / L
  UNROLL = 25
  assert L % UNROLL == 0

  mesh = plsc.VectorSubcoreMesh(core_axis_name="c", subcore_axis_name="s")

  @functools.partial(
      pl.kernel,
      out_type=jax.ShapeDtypeStruct((B, _LANES), jnp.float32),
      mesh=mesh,
      compiler_params=pltpu.CompilerParams(use_tc_tiling_on_sc=False),
      scratch_types=[
          pltpu.VMEM((BPW * SPLITS, _IDXW), jnp.int32),
          pltpu.VMEM((_NBUF * L, D), jnp.float32),
          pltpu.VMEM((D, _LANES), jnp.float32),
          pltpu.VMEM((_LANES,), jnp.float32),
          pltpu.VMEM((BPW, _LANES), jnp.float32),
      ] + [pltpu.SemaphoreType.DMA] * (_NBUF + 1),
  )
  def sc_kernel(text_hbm, table_hbm, fcwt_hbm, fcb_hbm, out_hbm,
                idx_v, rows_v, fcwt_v, fcb_v, out_v, *sems):
    wid = lax.axis_index("s") * NC + lax.axis_index("c")
    base = wid * BPW
    psem = sems[_NBUF]

    q = (BPW * SPLITS) // 4
    pltpu.async_copy(text_hbm.at[pl.ds(base * SPLITS, q)],
                     idx_v.at[pl.ds(0, q)], psem)
    pltpu.async_copy(fcwt_hbm, fcwt_v, psem)
    pltpu.async_copy(fcb_hbm, fcb_v, psem)
    pltpu.make_async_copy(text_hbm.at[pl.ds(0, q)],
                          idx_v.at[pl.ds(0, q)], psem).wait()
    pltpu.make_async_copy(fcwt_hbm, fcwt_v, psem).wait()
    pltpu.make_async_copy(fcb_hbm, fcb_v, psem).wait()
    pltpu.async_copy(text_hbm.at[pl.ds(base * SPLITS + q, 3 * q)],
                     idx_v.at[pl.ds(q, 3 * q)], psem)

    fcb_vec = fcb_v[pl.ds(0, _LANES)]
    zero = jnp.zeros((_LANES,), jnp.float32)

    def fire(r, b):
      for j in range(SPLITS):
        pltpu.async_copy(
            table_hbm.at[idx_v.at[r * SPLITS + j]],
            rows_v.at[pl.ds(b * L + j * _IDXW, _IDXW)], sems[b])

    def drain(b):
      pltpu.make_async_copy(table_hbm.at[pl.ds(0, L)],
                            rows_v.at[pl.ds(b * L, L)], sems[b]).wait()

    def process(r, b):
      off = b * L

      def red_body(t, accs):
        ib = off + t * UNROLL
        accs = list(accs)
        for u in range(UNROLL):
          for k in range(KD):
            accs[k] = accs[k] + rows_v[ib + u, pl.ds(k * _LANES, _LANES)]
        return tuple(accs)

      accs = lax.fori_loop(0, L // UNROLL, red_body, (zero,) * KD)
      pooled = [a * inv_l for a in accs]

      parts = [fcb_vec, zero, zero, zero]
      for d in range(D):
        parts[d % 4] = parts[d % 4] + (
            pooled[d // _LANES][d % _LANES] * fcwt_v[d, pl.ds(0, _LANES)])
      out_row = (parts[0] + parts[1]) + (parts[2] + parts[3])
      out_v[r, pl.ds(0, _LANES)] = out_row

    for b in range(_NBUF - 1):
      fire(b, b)
    pltpu.make_async_copy(text_hbm.at[pl.ds(0, 3 * q)],
                          idx_v.at[pl.ds(q, 3 * q)], psem).wait()

    def body(h, carry):
      for b in range(_NBUF):
        r = h * _NBUF + b
        nxt = r + _NBUF - 1
        pb = (b - 1) % _NBUF

        @pl.when(nxt < BPW)
        def _():
          fire(nxt, pb)

        drain(b)
        process(r, b)
      return carry

    lax.fori_loop(0, BPW // _NBUF, body, 0)
    pltpu.sync_copy(out_v, out_hbm.at[pl.ds(base, BPW)])

  return sc_kernel


def kernel(text, emb_table, fc_w, fc_b):
  B, L = text.shape
  V, D = emb_table.shape
  C = fc_w.shape[0]
  text = text.astype(jnp.int32).reshape(B * (L // _IDXW), _IDXW)
  fcwt = jnp.zeros((D, _LANES), jnp.float32).at[:, :C].set(fc_w.T)
  fcb_pad = jnp.zeros((_LANES,), jnp.float32).at[:C].set(fc_b)
  out = _build(B, L, V, D, C)(text, emb_table, fcwt, fcb_pad)
  return out[:, :C]
